# Initial kernel scaffold; baseline (speedup 1.0000x reference)
import jax
import jax.numpy as jnp
from jax.experimental import pallas as pl


def kernel(x1, edge_index, W1, b1, W2, b2, Wl1, bl1, Wl2, bl2):
    raise NotImplementedError("write your pallas kernel here")


# speedup vs baseline: 3.0106x; 3.0106x over previous
import functools

import jax
import jax.numpy as jnp
from jax.experimental import pallas as pl

_N = 7
_E = 14
_F0 = 16
_F1 = 8
_D0 = _N * _F0
_D1 = _N * _F1


def _eq_iota(shape, dim0_div, dim1_mod=None):
    r = jax.lax.broadcasted_iota(jnp.int32, shape, 0)
    c = jax.lax.broadcasted_iota(jnp.int32, shape, 1)
    if dim1_mod is None:
        return (r // dim0_div == c).astype(jnp.float32)
    return (r % dim1_mod == c).astype(jnp.float32)


def _prep_body(ei_ref, w1_ref, w2_ref, m1_ref, m2_ref):
    f32 = jnp.float32
    src = ei_ref[0:1, :]
    dst = ei_ref[1:2, :]
    rows = jax.lax.broadcasted_iota(jnp.int32, (_N, _E), 0)
    oh_src = (rows == src).astype(f32)
    oh_dst = (rows == dst).astype(f32)

    deg = jnp.sum(oh_dst, axis=1, keepdims=True) + 1.0
    dinv = jax.lax.rsqrt(deg)

    dot = functools.partial(
        jax.lax.dot_general, preferred_element_type=jnp.float32)
    cdim = (((1,), (1,)), ((), ()))

    dinv_src = jnp.sum(dinv * oh_src, axis=0, keepdims=True)
    dinv_dst = jnp.sum(dinv * oh_dst, axis=0, keepdims=True)
    norm = dinv_src * dinv_dst

    A = dot(oh_dst * norm, oh_src, cdim)
    eye_r = jax.lax.broadcasted_iota(jnp.int32, (_N, _N), 0)
    eye_c = jax.lax.broadcasted_iota(jnp.int32, (_N, _N), 1)
    A = A + (eye_r == eye_c).astype(f32) * (dinv * dinv)

    R16 = _eq_iota((_D0, _N), _F0)
    C8 = _eq_iota((_D1, _N), _F1)
    T16 = _eq_iota((_D0, _F0), 1, _F0)
    T8 = _eq_iota((_D1, _F1), 1, _F1)

    repA1 = dot(dot(R16, A, cdim), C8, cdim)
    tileW1 = dot(dot(T16, w1_ref[...], (((1,), (0,)), ((), ()))), T8, cdim)
    m1_ref[...] = repA1 * tileW1

    repA2 = dot(dot(C8, A, cdim), C8, cdim)
    tileW2 = dot(dot(T8, w2_ref[...], (((1,), (0,)), ((), ()))), T8, cdim)
    m2_ref[...] = repA2 * tileW2


def _main_body(x_ref, m1_ref, m2_ref, wl1_ref, wl2_ref,
               b1_ref, b2_ref, bl1_ref, bl2_ref, o_ref):
    dot = functools.partial(jnp.dot, preferred_element_type=jnp.float32)
    t = jnp.maximum(dot(x_ref[...], m1_ref[...]) + b1_ref[...], 0.0)
    t = jnp.maximum(dot(t, m2_ref[...]) + b2_ref[...], 0.0)
    t = jnp.maximum(dot(t, wl1_ref[...]) + bl1_ref[...], 0.0)
    o_ref[...] = dot(t, wl2_ref[...]) + bl2_ref[...]


def kernel(x1, edge_index, W1, b1, W2, b2, Wl1, bl1, Wl2, bl2):
    B = x1.shape[0]

    M1, M2 = pl.pallas_call(
        _prep_body,
        out_shape=(
            jax.ShapeDtypeStruct((_D0, _D1), jnp.float32),
            jax.ShapeDtypeStruct((_D1, _D1), jnp.float32),
        ),
    )(edge_index, W1, W2)

    b1t = jnp.tile(b1, _N).reshape(1, _D1)
    b2t = jnp.tile(b2, _N).reshape(1, _D1)
    bl1r = bl1.reshape(1, -1)
    bl2r = bl2.reshape(1, -1)

    T = 2048
    grid = (B // T,)
    out = pl.pallas_call(
        _main_body,
        grid=grid,
        in_specs=[
            pl.BlockSpec((T, _D0), lambda i: (i, 0)),
            pl.BlockSpec((_D0, _D1), lambda i: (0, 0)),
            pl.BlockSpec((_D1, _D1), lambda i: (0, 0)),
            pl.BlockSpec((_D1, 24), lambda i: (0, 0)),
            pl.BlockSpec((24, 1), lambda i: (0, 0)),
            pl.BlockSpec((1, _D1), lambda i: (0, 0)),
            pl.BlockSpec((1, _D1), lambda i: (0, 0)),
            pl.BlockSpec((1, 24), lambda i: (0, 0)),
            pl.BlockSpec((1, 1), lambda i: (0, 0)),
        ],
        out_specs=pl.BlockSpec((T, 1), lambda i: (i, 0)),
        out_shape=jax.ShapeDtypeStruct((B, 1), jnp.float32),
    )(x1, M1, M2, Wl1, Wl2, b1t, b2t, bl1r, bl2r)
    return out


# speedup vs baseline: 3.6625x; 1.2165x over previous
import functools

import jax
import jax.numpy as jnp
from jax.experimental import pallas as pl

_N = 7
_E = 14
_F0 = 16
_F1 = 8
_D0 = _N * _F0
_D1 = _N * _F1


def _eq_iota(shape, dim0_div, dim1_mod=None):
    r = jax.lax.broadcasted_iota(jnp.int32, shape, 0)
    c = jax.lax.broadcasted_iota(jnp.int32, shape, 1)
    if dim1_mod is None:
        return (r // dim0_div == c).astype(jnp.float32)
    return (r % dim1_mod == c).astype(jnp.float32)


def _prep_body(ei_ref, w1_ref, w2_ref, m1_ref, m2_ref):
    f32 = jnp.float32
    src = ei_ref[0:1, :]
    dst = ei_ref[1:2, :]
    rows = jax.lax.broadcasted_iota(jnp.int32, (_N, _E), 0)
    oh_src = (rows == src).astype(f32)
    oh_dst = (rows == dst).astype(f32)

    deg = jnp.sum(oh_dst, axis=1, keepdims=True) + 1.0
    dinv = jax.lax.rsqrt(deg)

    dot = functools.partial(
        jax.lax.dot_general, preferred_element_type=jnp.float32)
    cdim = (((1,), (1,)), ((), ()))

    dinv_src = jnp.sum(dinv * oh_src, axis=0, keepdims=True)
    dinv_dst = jnp.sum(dinv * oh_dst, axis=0, keepdims=True)
    norm = dinv_src * dinv_dst

    A = dot(oh_dst * norm, oh_src, cdim)
    eye_r = jax.lax.broadcasted_iota(jnp.int32, (_N, _N), 0)
    eye_c = jax.lax.broadcasted_iota(jnp.int32, (_N, _N), 1)
    A = A + (eye_r == eye_c).astype(f32) * (dinv * dinv)

    R16 = _eq_iota((_D0, _N), _F0)
    C8 = _eq_iota((_D1, _N), _F1)
    T16 = _eq_iota((_D0, _F0), 1, _F0)
    T8 = _eq_iota((_D1, _F1), 1, _F1)

    repA1 = dot(dot(R16, A, cdim), C8, cdim)
    tileW1 = dot(dot(T16, w1_ref[...], (((1,), (0,)), ((), ()))), T8, cdim)
    m1_ref[...] = repA1 * tileW1

    repA2 = dot(dot(C8, A, cdim), C8, cdim)
    tileW2 = dot(dot(T8, w2_ref[...], (((1,), (0,)), ((), ()))), T8, cdim)
    m2_ref[...] = repA2 * tileW2


def _main_body(x_ref, m1_ref, m2_ref, wl1_ref, wl2_ref,
               b1_ref, b2_ref, bl1_ref, bl2_ref, o_ref):
    dot = functools.partial(jnp.dot, preferred_element_type=jnp.float32)
    t = jnp.maximum(dot(x_ref[...], m1_ref[...]) + b1_ref[...], 0.0)
    t = jnp.maximum(dot(t, m2_ref[...]) + b2_ref[...], 0.0)
    t = jnp.maximum(dot(t, wl1_ref[...]) + bl1_ref[...], 0.0)
    o_ref[...] = dot(t, wl2_ref[...]) + bl2_ref[...]


def kernel(x1, edge_index, W1, b1, W2, b2, Wl1, bl1, Wl2, bl2):
    B = x1.shape[0]

    M1, M2 = pl.pallas_call(
        _prep_body,
        out_shape=(
            jax.ShapeDtypeStruct((_D0, _D1), jnp.float32),
            jax.ShapeDtypeStruct((_D1, _D1), jnp.float32),
        ),
    )(edge_index, W1, W2)

    b1t = jnp.tile(b1, _N).reshape(1, _D1)
    b2t = jnp.tile(b2, _N).reshape(1, _D1)
    bl1r = bl1.reshape(1, -1)
    bl2r = bl2.reshape(1, -1)

    T = 8192
    grid = (B // T,)
    out = pl.pallas_call(
        _main_body,
        grid=grid,
        in_specs=[
            pl.BlockSpec((T, _D0), lambda i: (i, 0)),
            pl.BlockSpec((_D0, _D1), lambda i: (0, 0)),
            pl.BlockSpec((_D1, _D1), lambda i: (0, 0)),
            pl.BlockSpec((_D1, 24), lambda i: (0, 0)),
            pl.BlockSpec((24, 1), lambda i: (0, 0)),
            pl.BlockSpec((1, _D1), lambda i: (0, 0)),
            pl.BlockSpec((1, _D1), lambda i: (0, 0)),
            pl.BlockSpec((1, 24), lambda i: (0, 0)),
            pl.BlockSpec((1, 1), lambda i: (0, 0)),
        ],
        out_specs=pl.BlockSpec((T, 1), lambda i: (i, 0)),
        out_shape=jax.ShapeDtypeStruct((B, 1), jnp.float32),
    )(x1, M1, M2, Wl1, Wl2, b1t, b2t, bl1r, bl2r)
    return out


# speedup vs baseline: 3.7599x; 1.0266x over previous
import functools

import jax
import jax.numpy as jnp
from jax.experimental import pallas as pl

_N = 7
_E = 14
_F0 = 16
_F1 = 8
_D0 = _N * _F0
_D1 = _N * _F1


def _eq_iota(shape, dim0_div, dim1_mod=None):
    r = jax.lax.broadcasted_iota(jnp.int32, shape, 0)
    c = jax.lax.broadcasted_iota(jnp.int32, shape, 1)
    if dim1_mod is None:
        return (r // dim0_div == c).astype(jnp.float32)
    return (r % dim1_mod == c).astype(jnp.float32)


def _prep_body(ei_ref, w1_ref, w2_ref, m1_ref, m2_ref):
    f32 = jnp.float32
    src = ei_ref[0:1, :]
    dst = ei_ref[1:2, :]
    rows = jax.lax.broadcasted_iota(jnp.int32, (_N, _E), 0)
    oh_src = (rows == src).astype(f32)
    oh_dst = (rows == dst).astype(f32)

    deg = jnp.sum(oh_dst, axis=1, keepdims=True) + 1.0
    dinv = jax.lax.rsqrt(deg)

    dot = functools.partial(
        jax.lax.dot_general, preferred_element_type=jnp.float32)
    cdim = (((1,), (1,)), ((), ()))

    dinv_src = jnp.sum(dinv * oh_src, axis=0, keepdims=True)
    dinv_dst = jnp.sum(dinv * oh_dst, axis=0, keepdims=True)
    norm = dinv_src * dinv_dst

    A = dot(oh_dst * norm, oh_src, cdim)
    eye_r = jax.lax.broadcasted_iota(jnp.int32, (_N, _N), 0)
    eye_c = jax.lax.broadcasted_iota(jnp.int32, (_N, _N), 1)
    A = A + (eye_r == eye_c).astype(f32) * (dinv * dinv)

    R16 = _eq_iota((_D0, _N), _F0)
    C8 = _eq_iota((_D1, _N), _F1)
    T16 = _eq_iota((_D0, _F0), 1, _F0)
    T8 = _eq_iota((_D1, _F1), 1, _F1)

    repA1 = dot(dot(R16, A, cdim), C8, cdim)
    tileW1 = dot(dot(T16, w1_ref[...], (((1,), (0,)), ((), ()))), T8, cdim)
    m1_ref[...] = repA1 * tileW1

    repA2 = dot(dot(C8, A, cdim), C8, cdim)
    tileW2 = dot(dot(T8, w2_ref[...], (((1,), (0,)), ((), ()))), T8, cdim)
    m2_ref[...] = repA2 * tileW2


def _main_body(x_ref, m1_ref, m2_ref, wl1_ref, wl2_ref,
               b1_ref, b2_ref, bl1_ref, bl2_ref, o_ref):
    dot = functools.partial(jnp.dot, preferred_element_type=jnp.float32)
    t = jnp.maximum(dot(x_ref[...], m1_ref[...]) + b1_ref[...], 0.0)
    t = jnp.maximum(dot(t, m2_ref[...]) + b2_ref[...], 0.0)
    t = jnp.maximum(dot(t, wl1_ref[...]) + bl1_ref[...], 0.0)
    o_ref[...] = dot(t, wl2_ref[...]) + bl2_ref[...]


def kernel(x1, edge_index, W1, b1, W2, b2, Wl1, bl1, Wl2, bl2):
    B = x1.shape[0]

    M1, M2 = pl.pallas_call(
        _prep_body,
        out_shape=(
            jax.ShapeDtypeStruct((_D0, _D1), jnp.float32),
            jax.ShapeDtypeStruct((_D1, _D1), jnp.float32),
        ),
    )(edge_index, W1, W2)

    b1t = jnp.tile(b1, _N).reshape(1, _D1)
    b2t = jnp.tile(b2, _N).reshape(1, _D1)
    bl1r = bl1.reshape(1, -1)
    bl2r = bl2.reshape(1, -1)

    T = 16384
    grid = (B // T,)
    out = pl.pallas_call(
        _main_body,
        grid=grid,
        in_specs=[
            pl.BlockSpec((T, _D0), lambda i: (i, 0)),
            pl.BlockSpec((_D0, _D1), lambda i: (0, 0)),
            pl.BlockSpec((_D1, _D1), lambda i: (0, 0)),
            pl.BlockSpec((_D1, 24), lambda i: (0, 0)),
            pl.BlockSpec((24, 1), lambda i: (0, 0)),
            pl.BlockSpec((1, _D1), lambda i: (0, 0)),
            pl.BlockSpec((1, _D1), lambda i: (0, 0)),
            pl.BlockSpec((1, 24), lambda i: (0, 0)),
            pl.BlockSpec((1, 1), lambda i: (0, 0)),
        ],
        out_specs=pl.BlockSpec((T, 1), lambda i: (i, 0)),
        out_shape=jax.ShapeDtypeStruct((B, 1), jnp.float32),
    )(x1, M1, M2, Wl1, Wl2, b1t, b2t, bl1r, bl2r)
    return out


# speedup vs baseline: 3.8545x; 1.0252x over previous
import functools

import jax
import jax.numpy as jnp
from jax.experimental import pallas as pl
from jax.experimental.pallas import tpu as pltpu

_N = 7
_E = 14
_F0 = 16
_F1 = 8
_D0 = _N * _F0
_D1 = _N * _F1

_dot = functools.partial(
    jax.lax.dot_general, preferred_element_type=jnp.float32)
_C11 = (((1,), (1,)), ((), ()))
_C10 = (((1,), (0,)), ((), ()))


def _eq_iota(shape, div, mod=None):
    r = jax.lax.broadcasted_iota(jnp.int32, shape, 0)
    c = jax.lax.broadcasted_iota(jnp.int32, shape, 1)
    if mod is None:
        return (r // div == c).astype(jnp.float32)
    return (r % mod == c).astype(jnp.float32)


def _prep(ei_ref, w1_ref, w2_ref, b1_ref, b2_ref,
          m1_scr, m2_scr, b1t_scr, b2t_scr):
    f32 = jnp.float32
    src = ei_ref[0:1, :]
    dst = ei_ref[1:2, :]
    rows = jax.lax.broadcasted_iota(jnp.int32, (_N, _E), 0)
    oh_src = (rows == src).astype(f32)
    oh_dst = (rows == dst).astype(f32)

    deg = jnp.sum(oh_dst, axis=1, keepdims=True) + 1.0
    dinv = jax.lax.rsqrt(deg)

    dinv_src = jnp.sum(dinv * oh_src, axis=0, keepdims=True)
    dinv_dst = jnp.sum(dinv * oh_dst, axis=0, keepdims=True)
    norm = dinv_src * dinv_dst

    A = _dot(oh_dst * norm, oh_src, _C11)
    eye_r = jax.lax.broadcasted_iota(jnp.int32, (_N, _N), 0)
    eye_c = jax.lax.broadcasted_iota(jnp.int32, (_N, _N), 1)
    A = A + (eye_r == eye_c).astype(f32) * (dinv * dinv)

    R16 = _eq_iota((_D0, _N), _F0)
    C8 = _eq_iota((_D1, _N), _F1)
    T16 = _eq_iota((_D0, _F0), 1, _F0)
    T8 = _eq_iota((_D1, _F1), 1, _F1)

    repA1 = _dot(_dot(R16, A, _C11), C8, _C11)
    tileW1 = _dot(_dot(T16, w1_ref[...], _C10), T8, _C11)
    m1_scr[...] = repA1 * tileW1

    repA2 = _dot(_dot(C8, A, _C11), C8, _C11)
    tileW2 = _dot(_dot(T8, w2_ref[...], _C10), T8, _C11)
    m2_scr[...] = repA2 * tileW2

    b1t_scr[...] = _dot(b1_ref[...], T8, _C11)
    b2t_scr[...] = _dot(b2_ref[...], T8, _C11)


def _body(ei_ref, w1_ref, w2_ref, b1_ref, b2_ref,
          wl1_ref, wl2_ref, bl1_ref, bl2_ref, x_ref, o_ref,
          m1_scr, m2_scr, b1t_scr, b2t_scr):
    @pl.when(pl.program_id(0) == 0)
    def _():
        _prep(ei_ref, w1_ref, w2_ref, b1_ref, b2_ref,
              m1_scr, m2_scr, b1t_scr, b2t_scr)

    t = jnp.maximum(_dot(x_ref[...], m1_scr[...], _C10) + b1t_scr[...], 0.0)
    t = jnp.maximum(_dot(t, m2_scr[...], _C10) + b2t_scr[...], 0.0)
    t = jnp.maximum(_dot(t, wl1_ref[...], _C10) + bl1_ref[...], 0.0)
    o_ref[...] = _dot(t, wl2_ref[...], _C10) + bl2_ref[...]


def kernel(x1, edge_index, W1, b1, W2, b2, Wl1, bl1, Wl2, bl2):
    B = x1.shape[0]
    T = 16384
    rep = lambda i: (0, 0)
    out = pl.pallas_call(
        _body,
        grid=(B // T,),
        in_specs=[
            pl.BlockSpec((2, _E), rep),
            pl.BlockSpec((_F0, _F1), rep),
            pl.BlockSpec((_F1, _F1), rep),
            pl.BlockSpec((1, _F1), rep),
            pl.BlockSpec((1, _F1), rep),
            pl.BlockSpec((_D1, 24), rep),
            pl.BlockSpec((24, 1), rep),
            pl.BlockSpec((1, 24), rep),
            pl.BlockSpec((1, 1), rep),
            pl.BlockSpec((T, _D0), lambda i: (i, 0)),
        ],
        out_specs=pl.BlockSpec((T, 1), lambda i: (i, 0)),
        out_shape=jax.ShapeDtypeStruct((B, 1), jnp.float32),
        scratch_shapes=[
            pltpu.VMEM((_D0, _D1), jnp.float32),
            pltpu.VMEM((_D1, _D1), jnp.float32),
            pltpu.VMEM((1, _D1), jnp.float32),
            pltpu.VMEM((1, _D1), jnp.float32),
        ],
    )(edge_index, W1, W2, b1.reshape(1, -1), b2.reshape(1, -1),
      Wl1, Wl2, bl1.reshape(1, -1), bl2.reshape(1, -1), x1)
    return out
